# Initial kernel scaffold; baseline (speedup 1.0000x reference)
import jax
import jax.numpy as jnp
from jax.experimental import pallas as pl


def kernel(features, input_mask):
    raise NotImplementedError("write your pallas kernel here")


# speedup vs baseline: 1.5065x; 1.5065x over previous
import jax
import jax.numpy as jnp
from jax.experimental import pallas as pl
from jax.experimental.pallas import tpu as pltpu

_LANE = 128
_SUBLANE = 8


def _pool_kernel(feat_ref, mask_ref, out_ref):
    x = feat_ref[...].astype(jnp.float32)
    m = mask_ref[...]
    acc = jnp.sum(x * m, axis=1)
    cnt = jnp.sum(m, axis=1)
    inv = 1.0 / jnp.maximum(cnt, 1e-9)
    out_ref[...] = (acc * inv).astype(out_ref.dtype)


def kernel(features, input_mask):
    B, S, H = features.shape
    TB = _SUBLANE if B % _SUBLANE == 0 else B
    nb = B // TB if B % _SUBLANE == 0 else 1

    mask3 = input_mask.astype(jnp.float32).reshape(B, S, 1)

    vmem_limit = 60 * 1024 * 1024
    return pl.pallas_call(
        _pool_kernel,
        out_shape=jax.ShapeDtypeStruct((B, H), features.dtype),
        grid=(nb,),
        in_specs=[
            pl.BlockSpec((TB, S, H), lambda b: (b, 0, 0)),
            pl.BlockSpec((TB, S, 1), lambda b: (b, 0, 0)),
        ],
        out_specs=pl.BlockSpec((TB, H), lambda b: (b, 0)),
        compiler_params=pltpu.CompilerParams(
            dimension_semantics=("parallel",),
            vmem_limit_bytes=vmem_limit,
        ),
        cost_estimate=pl.CostEstimate(
            flops=2 * B * S * H,
            transcendentals=0,
            bytes_accessed=B * S * H * 4 + B * S * 4 + B * H * 4,
        ),
    )(features, mask3)


# speedup vs baseline: 1.8194x; 1.2077x over previous
import functools

import jax
import jax.numpy as jnp
from jax.experimental import pallas as pl
from jax.experimental.pallas import tpu as pltpu

_LANE = 128
_SUBLANE = 8


def _pool_kernel(mask_ref, feat_ref, out_ref, acc_ref, cnt_ref, *, tb, ns):
    s = pl.program_id(1)

    @pl.when(s == 0)
    def _init():
        acc_ref[...] = jnp.zeros_like(acc_ref)
        cnt_ref[...] = jnp.zeros_like(cnt_ref)

    m = mask_ref[...]
    x = feat_ref[...]
    cnt_ref[...] += jnp.sum(m, axis=1, keepdims=True)
    rows = [
        jax.lax.dot_general(
            m[i : i + 1, :], x[i],
            dimension_numbers=(((1,), (0,)), ((), ())),
            preferred_element_type=jnp.float32,
        )
        for i in range(tb)
    ]
    acc_ref[...] += jnp.concatenate(rows, axis=0)

    @pl.when(s == ns - 1)
    def _finalize():
        inv = 1.0 / jnp.maximum(cnt_ref[...], 1e-9)
        out_ref[...] = (acc_ref[...] * inv).astype(out_ref.dtype)


def kernel(features, input_mask):
    B, S, H = features.shape
    TB = _SUBLANE if B % _SUBLANE == 0 else B
    nb = B // TB if B % _SUBLANE == 0 else 1
    TS = _LANE if S % _LANE == 0 else S
    ns = S // TS if S % _LANE == 0 else 1

    mask = input_mask.astype(jnp.float32)

    body = functools.partial(_pool_kernel, tb=TB, ns=ns)
    return pl.pallas_call(
        body,
        out_shape=jax.ShapeDtypeStruct((B, H), features.dtype),
        grid=(nb, ns),
        in_specs=[
            pl.BlockSpec((TB, TS), lambda b, s: (b, s)),
            pl.BlockSpec((TB, TS, H), lambda b, s: (b, s, 0)),
        ],
        out_specs=pl.BlockSpec((TB, H), lambda b, s: (b, 0)),
        scratch_shapes=[
            pltpu.VMEM((TB, H), jnp.float32),
            pltpu.VMEM((TB, 1), jnp.float32),
        ],
        compiler_params=pltpu.CompilerParams(
            dimension_semantics=("parallel", "arbitrary"),
            vmem_limit_bytes=60 * 1024 * 1024,
        ),
        cost_estimate=pl.CostEstimate(
            flops=2 * B * S * H,
            transcendentals=0,
            bytes_accessed=B * S * H * 4 + B * S * 4 + B * H * 4,
        ),
    )(mask, features)


# speedup vs baseline: 1.9798x; 1.0881x over previous
import functools

import jax
import jax.numpy as jnp
from jax.experimental import pallas as pl
from jax.experimental.pallas import tpu as pltpu

_LANE = 128
_SUBLANE = 8


def _pool_kernel(mask_ref, feat_ref, out_ref, acc_ref, cnt_ref, *, tb, ns):
    s = pl.program_id(1)

    @pl.when(s == 0)
    def _init():
        acc_ref[...] = jnp.zeros_like(acc_ref)
        cnt_ref[...] = jnp.zeros_like(cnt_ref)

    m = mask_ref[...]
    x = feat_ref[...]
    cnt_ref[...] += jnp.sum(m, axis=1, keepdims=True)
    rows = [
        jax.lax.dot_general(
            m[i : i + 1, :], x[i],
            dimension_numbers=(((1,), (0,)), ((), ())),
            preferred_element_type=jnp.float32,
        )
        for i in range(tb)
    ]
    acc_ref[...] += jnp.concatenate(rows, axis=0)

    @pl.when(s == ns - 1)
    def _finalize():
        inv = 1.0 / jnp.maximum(cnt_ref[...], 1e-9)
        out_ref[...] = (acc_ref[...] * inv).astype(out_ref.dtype)


def kernel(features, input_mask):
    B, S, H = features.shape
    TB = _SUBLANE if B % _SUBLANE == 0 else B
    nb = B // TB if B % _SUBLANE == 0 else 1
    TS = 2 * _LANE if S % (2 * _LANE) == 0 else S
    ns = S // TS if S % _LANE == 0 else 1

    mask = input_mask.astype(jnp.float32)

    body = functools.partial(_pool_kernel, tb=TB, ns=ns)
    return pl.pallas_call(
        body,
        out_shape=jax.ShapeDtypeStruct((B, H), features.dtype),
        grid=(nb, ns),
        in_specs=[
            pl.BlockSpec((TB, TS), lambda b, s: (b, s)),
            pl.BlockSpec((TB, TS, H), lambda b, s: (b, s, 0)),
        ],
        out_specs=pl.BlockSpec((TB, H), lambda b, s: (b, 0)),
        scratch_shapes=[
            pltpu.VMEM((TB, H), jnp.float32),
            pltpu.VMEM((TB, 1), jnp.float32),
        ],
        compiler_params=pltpu.CompilerParams(
            dimension_semantics=("parallel", "arbitrary"),
            vmem_limit_bytes=60 * 1024 * 1024,
        ),
        cost_estimate=pl.CostEstimate(
            flops=2 * B * S * H,
            transcendentals=0,
            bytes_accessed=B * S * H * 4 + B * S * 4 + B * H * 4,
        ),
    )(mask, features)


# speedup vs baseline: 1.9982x; 1.0093x over previous
import functools

import jax
import jax.numpy as jnp
from jax.experimental import pallas as pl
from jax.experimental.pallas import tpu as pltpu

_LANE = 128
_SUBLANE = 8


def _pool_kernel(mask_ref, feat_ref, out_ref, acc_ref, cnt_ref, *, tb, ns):
    s = pl.program_id(1)

    @pl.when(s == 0)
    def _init():
        acc_ref[...] = jnp.zeros_like(acc_ref)
        cnt_ref[...] = jnp.zeros_like(cnt_ref)

    m = mask_ref[...]
    x = feat_ref[...]
    cnt_ref[...] += jnp.sum(m, axis=1, keepdims=True)
    rows = [
        jax.lax.dot_general(
            m[i : i + 1, :], x[i],
            dimension_numbers=(((1,), (0,)), ((), ())),
            preferred_element_type=jnp.float32,
        )
        for i in range(tb)
    ]
    acc_ref[...] += jnp.concatenate(rows, axis=0)

    @pl.when(s == ns - 1)
    def _finalize():
        inv = 1.0 / jnp.maximum(cnt_ref[...], 1e-9)
        out_ref[...] = (acc_ref[...] * inv).astype(out_ref.dtype)


def kernel(features, input_mask):
    B, S, H = features.shape
    TB = _SUBLANE if B % _SUBLANE == 0 else B
    nb = B // TB if B % _SUBLANE == 0 else 1
    TS = 4 * _LANE if S % (4 * _LANE) == 0 else S
    ns = S // TS if S % _LANE == 0 else 1

    mask = input_mask.astype(jnp.float32)

    body = functools.partial(_pool_kernel, tb=TB, ns=ns)
    return pl.pallas_call(
        body,
        out_shape=jax.ShapeDtypeStruct((B, H), features.dtype),
        grid=(nb, ns),
        in_specs=[
            pl.BlockSpec((TB, TS), lambda b, s: (b, s)),
            pl.BlockSpec((TB, TS, H), lambda b, s: (b, s, 0)),
        ],
        out_specs=pl.BlockSpec((TB, H), lambda b, s: (b, 0)),
        scratch_shapes=[
            pltpu.VMEM((TB, H), jnp.float32),
            pltpu.VMEM((TB, 1), jnp.float32),
        ],
        compiler_params=pltpu.CompilerParams(
            dimension_semantics=("parallel", "arbitrary"),
            vmem_limit_bytes=60 * 1024 * 1024,
        ),
        cost_estimate=pl.CostEstimate(
            flops=2 * B * S * H,
            transcendentals=0,
            bytes_accessed=B * S * H * 4 + B * S * 4 + B * H * 4,
        ),
    )(mask, features)
